# Initial kernel scaffold; baseline (speedup 1.0000x reference)
import jax
import jax.numpy as jnp
from jax.experimental import pallas as pl


def kernel(input, span_idxs):
    raise NotImplementedError("write your pallas kernel here")


# speedup vs baseline: 1.6405x; 1.6405x over previous
import jax
import jax.numpy as jnp
from jax import lax
from jax.experimental import pallas as pl
from jax.experimental.pallas import tpu as pltpu
from jax.experimental.pallas import tpu_sc as plsc

B = 4
T = 2048
H = 512
N = 256
NSPAN = B * N
OUT_D = 4 * H

NC = 2
NS = 16
NW = NC * NS
SPW = NSPAN // NW
CH = 16
NCHUNK = SPW // CH
L = 16
G = H // L


def _body(table_hbm, i_hbm, j_hbm, out_hbm, iv, jv, idx_v, mult_v, rows_v,
          out_v, sem):
  wid = lax.axis_index("s") * NC + lax.axis_index("c")
  base = wid * SPW
  row_base = (wid // (N // SPW)) * (2 * T)

  pltpu.sync_copy(i_hbm.at[pl.ds(base, SPW)], iv)
  pltpu.sync_copy(j_hbm.at[pl.ds(base, SPW)], jv)

  for c in range(NCHUNK):
    i16 = iv[pl.ds(c * CH, L)]
    j16 = jv[pl.ds(c * CH, L)]

    fend_idx = 2 * j16 + row_base
    fpre_idx = 2 * jnp.maximum(i16 - 1, 0) + row_base
    bsta_idx = 2 * i16 + 1 + row_base
    bpos_idx = 2 * jnp.minimum(j16 + 1, T - 1) + 1 + row_base

    one = jnp.full((L,), 1.0, jnp.float32)
    zero = jnp.zeros((L,), jnp.float32)
    k16 = jnp.where((i16 != 0) | (j16 != 0), one, zero)
    a16 = jnp.where(i16 >= 1, k16, zero)
    c16 = jnp.where(j16 < T - 1, k16, zero)

    idx_v[pl.ds(0 * CH, L)] = fend_idx
    idx_v[pl.ds(1 * CH, L)] = fpre_idx
    idx_v[pl.ds(2 * CH, L)] = bsta_idx
    idx_v[pl.ds(3 * CH, L)] = bpos_idx
    mult_v[pl.ds(0 * CH, L)] = k16
    mult_v[pl.ds(1 * CH, L)] = a16
    mult_v[pl.ds(2 * CH, L)] = c16

    pltpu.async_copy(table_hbm.at[idx_v], rows_v, sem).wait()

    def span_body(s, _):
      sidx = jnp.full((L,), s, jnp.int32)
      kk = plsc.load_gather(mult_v, [sidx])
      aa = plsc.load_gather(mult_v, [sidx + CH])
      cc = plsc.load_gather(mult_v, [sidx + 2 * CH])

      def grp_body(g, _):
        off = g * L
        fend = rows_v[s, pl.ds(off, L)]
        fpre = rows_v[CH + s, pl.ds(off, L)]
        bsta = rows_v[2 * CH + s, pl.ds(off, L)]
        bpos = rows_v[3 * CH + s, pl.ds(off, L)]
        fpa = fpre * aa
        bpc = bpos * cc
        out_v[s, pl.ds(off, L)] = fend * kk - fpa
        out_v[s, pl.ds(H + off, L)] = bsta * kk - bpc
        out_v[s, pl.ds(2 * H + off, L)] = fpa
        out_v[s, pl.ds(3 * H + off, L)] = bpc
        return 0

      lax.fori_loop(0, G, grp_body, 0, unroll=4)
      return 0

    lax.fori_loop(0, CH, span_body, 0)

    pltpu.sync_copy(out_v, out_hbm.at[pl.ds(base + c * CH, CH)])


@jax.jit
def _launch(table, i_flat, j_flat):
  mesh = plsc.VectorSubcoreMesh(core_axis_name="c", subcore_axis_name="s")
  return pl.kernel(
      _body,
      out_type=jax.ShapeDtypeStruct((NSPAN, OUT_D), jnp.float32),
      mesh=mesh,
      compiler_params=pltpu.CompilerParams(needs_layout_passes=False),
      scratch_types=[
          pltpu.VMEM((SPW,), jnp.int32),
          pltpu.VMEM((SPW,), jnp.int32),
          pltpu.VMEM((4 * CH,), jnp.int32),
          pltpu.VMEM((3 * CH,), jnp.float32),
          pltpu.VMEM((4 * CH, H), jnp.float32),
          pltpu.VMEM((CH, OUT_D), jnp.float32),
          pltpu.SemaphoreType.DMA,
      ],
  )(table, i_flat, j_flat)


def kernel(input, span_idxs):
  table = input.reshape(B * T * 2, H)
  ij = span_idxs.reshape(NSPAN, 2)
  i_flat = ij[:, 0].astype(jnp.int32)
  j_flat = ij[:, 1].astype(jnp.int32)
  out = _launch(table, i_flat, j_flat)
  return out.reshape(B, N, OUT_D)


# speedup vs baseline: 3.0961x; 1.8873x over previous
import jax
import jax.numpy as jnp
from jax import lax
from jax.experimental import pallas as pl
from jax.experimental.pallas import tpu as pltpu
from jax.experimental.pallas import tpu_sc as plsc

B = 4
T = 2048
H = 512
N = 256
NSPAN = B * N
OUT_D = 4 * H

NC = 2
NS = 16
NW = NC * NS
SPW = NSPAN // NW
CH = 16
NCHUNK = SPW // CH
L = 16
G = H // L


def _body(x_hbm, i_hbm, j_hbm, out_hbm, iv, jv, ia_v, ib_v, ic_v, id_v,
          mult_v, rows_v, out_v, sem):
  wid = lax.axis_index("s") * NC + lax.axis_index("c")
  base = wid * SPW
  row_base = (wid // (N // SPW)) * T

  pltpu.sync_copy(i_hbm.at[pl.ds(base, SPW)], iv)
  pltpu.sync_copy(j_hbm.at[pl.ds(base, SPW)], jv)

  for c in range(NCHUNK):
    i16 = iv[pl.ds(c * CH, L)]
    j16 = jv[pl.ds(c * CH, L)]

    one = jnp.full((L,), 1.0, jnp.float32)
    zero = jnp.zeros((L,), jnp.float32)
    k16 = jnp.where((i16 != 0) | (j16 != 0), one, zero)
    a16 = jnp.where(i16 >= 1, k16, zero)
    c16 = jnp.where(j16 < T - 1, k16, zero)

    ia_v[...] = j16 + row_base
    ib_v[...] = jnp.maximum(i16 - 1, 0) + row_base
    ic_v[...] = i16 + row_base
    id_v[...] = jnp.minimum(j16 + 1, T - 1) + row_base
    mult_v[pl.ds(0 * CH, L)] = k16
    mult_v[pl.ds(1 * CH, L)] = a16
    mult_v[pl.ds(2 * CH, L)] = c16

    cp_a = pltpu.async_copy(x_hbm.at[ia_v, pl.ds(0, H)],
                            rows_v.at[pl.ds(0 * CH, CH)], sem)
    cp_b = pltpu.async_copy(x_hbm.at[ib_v, pl.ds(0, H)],
                            rows_v.at[pl.ds(1 * CH, CH)], sem)
    cp_c = pltpu.async_copy(x_hbm.at[ic_v, pl.ds(H, H)],
                            rows_v.at[pl.ds(2 * CH, CH)], sem)
    cp_d = pltpu.async_copy(x_hbm.at[id_v, pl.ds(H, H)],
                            rows_v.at[pl.ds(3 * CH, CH)], sem)
    cp_a.wait()
    cp_b.wait()
    cp_c.wait()
    cp_d.wait()

    def span_body(s, _):
      sidx = jnp.full((L,), s, jnp.int32)
      kk = plsc.load_gather(mult_v, [sidx])
      aa = plsc.load_gather(mult_v, [sidx + CH])
      cc = plsc.load_gather(mult_v, [sidx + 2 * CH])

      def grp_body(g, _):
        off = g * L
        fend = rows_v[s, pl.ds(off, L)]
        fpre = rows_v[CH + s, pl.ds(off, L)]
        bsta = rows_v[2 * CH + s, pl.ds(off, L)]
        bpos = rows_v[3 * CH + s, pl.ds(off, L)]
        fpa = fpre * aa
        bpc = bpos * cc
        out_v[s, pl.ds(off, L)] = fend * kk - fpa
        out_v[s, pl.ds(H + off, L)] = bsta * kk - bpc
        out_v[s, pl.ds(2 * H + off, L)] = fpa
        out_v[s, pl.ds(3 * H + off, L)] = bpc
        return 0

      lax.fori_loop(0, G, grp_body, 0, unroll=4)
      return 0

    lax.fori_loop(0, CH, span_body, 0)

    pltpu.sync_copy(out_v, out_hbm.at[pl.ds(base + c * CH, CH)])


@jax.jit
def _launch(x2, i_flat, j_flat):
  mesh = plsc.VectorSubcoreMesh(core_axis_name="c", subcore_axis_name="s")
  return pl.kernel(
      _body,
      out_type=jax.ShapeDtypeStruct((NSPAN, OUT_D), jnp.float32),
      mesh=mesh,
      compiler_params=pltpu.CompilerParams(needs_layout_passes=False),
      scratch_types=[
          pltpu.VMEM((SPW,), jnp.int32),
          pltpu.VMEM((SPW,), jnp.int32),
          pltpu.VMEM((CH,), jnp.int32),
          pltpu.VMEM((CH,), jnp.int32),
          pltpu.VMEM((CH,), jnp.int32),
          pltpu.VMEM((CH,), jnp.int32),
          pltpu.VMEM((3 * CH,), jnp.float32),
          pltpu.VMEM((4 * CH, H), jnp.float32),
          pltpu.VMEM((CH, OUT_D), jnp.float32),
          pltpu.SemaphoreType.DMA,
      ],
  )(x2, i_flat, j_flat)


def kernel(input, span_idxs):
  x2 = input.reshape(B * T, 2 * H)
  ij = span_idxs.reshape(NSPAN, 2)
  i_flat = ij[:, 0].astype(jnp.int32)
  j_flat = ij[:, 1].astype(jnp.int32)
  out = _launch(x2, i_flat, j_flat)
  return out.reshape(B, N, OUT_D)
